# Initial kernel scaffold; baseline (speedup 1.0000x reference)
import jax
import jax.numpy as jnp
from jax.experimental import pallas as pl


def kernel(x, angle):
    raise NotImplementedError("write your pallas kernel here")


# speedup vs baseline: 652.1606x; 652.1606x over previous
import jax
import jax.numpy as jnp
from jax.experimental import pallas as pl

_D = 8192
_R = 64
_CL = 128


def _crx_kernel(ang_ref, x_ref, re_ref, im_ref):
    half = ang_ref[0] * 0.5
    c = jnp.cos(half)
    s = jnp.sin(half)
    lo = x_ref[0:32, :]
    t0 = x_ref[32:48, :]
    t1 = x_ref[48:64, :]
    re_ref[0:32, :] = lo
    re_ref[32:48, :] = c * t0
    re_ref[48:64, :] = c * t1
    im_ref[0:32, :] = jnp.zeros_like(lo)
    im_ref[32:48, :] = (-s) * t1
    im_ref[48:64, :] = (-s) * t0


def kernel(x, angle):
    x2 = x.reshape(_R, _CL)
    re, im = pl.pallas_call(
        _crx_kernel,
        out_shape=(
            jax.ShapeDtypeStruct((_R, _CL), jnp.float32),
            jax.ShapeDtypeStruct((_R, _CL), jnp.float32),
        ),
    )(angle, x2)
    return jax.lax.complex(re, im).reshape(_D, 1)
